# initial kernel scaffold (unmeasured)
import jax
import jax.numpy as jnp
from jax import lax
from jax.experimental import pallas as pl
from jax.experimental.pallas import tpu as pltpu


def kernel(
    x,
):
    def body(*refs):
        pass

    out_shape = jax.ShapeDtypeStruct(..., jnp.float32)
    return pl.pallas_call(body, out_shape=out_shape)(...)


# baseline (device time: 18463 ns/iter reference)
import jax
import jax.numpy as jnp
from jax import lax
from jax.experimental import pallas as pl
from jax.experimental.pallas import tpu as pltpu

X_SIZE = 2


def kernel(x):
    m_per, n = x.shape
    n_half = n // X_SIZE

    def body(x_ref, out_ref, send_buf, send_sem, recv_sem):
        p = lax.axis_index("x")
        my_y = lax.axis_index("y")
        my_z = lax.axis_index("z")
        partner = 1 - p

        barrier_sem = pltpu.get_barrier_semaphore()
        pl.semaphore_signal(
            barrier_sem,
            inc=1,
            device_id=(partner, my_y, my_z),
            device_id_type=pl.DeviceIdType.MESH,
        )
        pl.semaphore_wait(barrier_sem, 1)

        send_buf[...] = x_ref[:, pl.ds(partner * n_half, n_half)].astype(
            jnp.bfloat16
        )

        rdma = pltpu.make_async_remote_copy(
            src_ref=send_buf,
            dst_ref=out_ref.at[pl.ds(p * m_per, m_per)],
            send_sem=send_sem,
            recv_sem=recv_sem,
            device_id=(partner, my_y, my_z),
            device_id_type=pl.DeviceIdType.MESH,
        )
        rdma.start()

        out_ref[pl.ds(p * m_per, m_per)] = x_ref[
            :, pl.ds(p * n_half, n_half)
        ].astype(jnp.bfloat16)

        rdma.wait()

    return pl.pallas_call(
        body,
        out_shape=jax.ShapeDtypeStruct((X_SIZE * m_per, n_half), jnp.bfloat16),
        in_specs=[pl.BlockSpec(memory_space=pltpu.VMEM)],
        out_specs=pl.BlockSpec(memory_space=pltpu.VMEM),
        scratch_shapes=[
            pltpu.VMEM((m_per, n_half), jnp.bfloat16),
            pltpu.SemaphoreType.DMA,
            pltpu.SemaphoreType.DMA,
        ],
        compiler_params=pltpu.CompilerParams(collective_id=0),
    )(x)


# device time: 16650 ns/iter; 1.1089x vs baseline; 1.1089x over previous
import jax
import jax.numpy as jnp
from jax import lax
from jax.experimental import pallas as pl
from jax.experimental.pallas import tpu as pltpu

M = 1024
N = 1024
NH = 512
H = 512
C = 4
CH = H // C


def kernel(x):
    def body(x_ref, out_ref, send_buf, s1_send, s1_recv, s2_send, s2_recv):
        p = lax.axis_index("x")
        y = lax.axis_index("y")
        z = lax.axis_index("z")
        q = 1 - p
        r = y % 2
        by = y + 1 - 2 * r

        barrier_sem = pltpu.get_barrier_semaphore()
        pl.semaphore_signal(
            barrier_sem, inc=1, device_id=(q, y, z),
            device_id_type=pl.DeviceIdType.MESH,
        )
        pl.semaphore_signal(
            barrier_sem, inc=1, device_id=(p, by, z),
            device_id_type=pl.DeviceIdType.MESH,
        )
        pl.semaphore_wait(barrier_sem, 2)

        rdma1 = []
        for c in range(C):
            row0 = r * H + c * CH

            @pl.when(p == 0)
            def _(row0=row0, c=c):
                send_buf[pl.ds(c * CH, CH)] = x_ref[
                    pl.ds(row0, CH), NH:N
                ].astype(jnp.bfloat16)

            @pl.when(p == 1)
            def _(row0=row0, c=c):
                send_buf[pl.ds(c * CH, CH)] = x_ref[
                    pl.ds(row0, CH), 0:NH
                ].astype(jnp.bfloat16)

            rdma = pltpu.make_async_remote_copy(
                src_ref=send_buf.at[pl.ds(c * CH, CH)],
                dst_ref=out_ref.at[pl.ds(p * M + row0, CH)],
                send_sem=s1_send.at[c],
                recv_sem=s1_recv.at[c],
                device_id=(q, y, z),
                device_id_type=pl.DeviceIdType.MESH,
            )
            rdma.start()
            rdma1.append(rdma)

        @pl.when(p == 0)
        def _():
            out_ref[0:M] = x_ref[:, 0:NH].astype(jnp.bfloat16)

        @pl.when(p == 1)
        def _():
            out_ref[M : 2 * M] = x_ref[:, NH:N].astype(jnp.bfloat16)

        rdma2 = []
        for c in range(C):
            row0 = q * M + r * H + c * CH
            rdma1[c].wait_recv()
            rdma = pltpu.make_async_remote_copy(
                src_ref=out_ref.at[pl.ds(row0, CH)],
                dst_ref=out_ref.at[pl.ds(row0, CH)],
                send_sem=s2_send.at[c],
                recv_sem=s2_recv.at[c],
                device_id=(p, by, z),
                device_id_type=pl.DeviceIdType.MESH,
            )
            rdma.start()
            rdma2.append(rdma)

        for c in range(C):
            rdma1[c].wait_send()
            rdma2[c].wait()

    return pl.pallas_call(
        body,
        out_shape=jax.ShapeDtypeStruct((2 * M, NH), jnp.bfloat16),
        in_specs=[pl.BlockSpec(memory_space=pltpu.VMEM)],
        out_specs=pl.BlockSpec(memory_space=pltpu.VMEM),
        scratch_shapes=[
            pltpu.VMEM((H, NH), jnp.bfloat16),
            pltpu.SemaphoreType.DMA((C,)),
            pltpu.SemaphoreType.DMA((C,)),
            pltpu.SemaphoreType.DMA((C,)),
            pltpu.SemaphoreType.DMA((C,)),
        ],
        compiler_params=pltpu.CompilerParams(collective_id=0),
    )(x)
